# baseline (speedup 1.0000x reference)
import functools

import jax
import jax.numpy as jnp
from jax import lax
from jax.experimental import pallas as pl
from jax.experimental.pallas import tpu as pltpu
from jax.experimental.pallas import tpu_sc as plsc

_VOCAB = 1000000
_EMBED = 64
_B = 4096
_L = 200
_N = _B * _L
_M = _N // 2

_GATHER_WINDOW = 128


def _sc_gather(token_table, flat_ids):
    mesh = plsc.VectorSubcoreMesh(core_axis_name="c", subcore_axis_name="s")

    @functools.partial(
        pl.kernel,
        out_type=jax.ShapeDtypeStruct((_N, _EMBED), jnp.float32),
        mesh=mesh,
        compiler_params=pltpu.CompilerParams(use_tc_tiling_on_sc=False),
    )
    def gather_kernel(table_hbm, idx_hbm, out_hbm):
        def body(i_vmem, o_vmem):
            pltpu.sync_copy(table_hbm.at[i_vmem.at[0]], o_vmem)

        pltpu.emit_pipeline(
            body,
            grid=(_N // _GATHER_WINDOW,),
            in_specs=[
                pl.BlockSpec((1, _GATHER_WINDOW), index_map=lambda i: (0, i))
            ],
            out_specs=[
                pl.BlockSpec((_GATHER_WINDOW, _EMBED), index_map=lambda i: (i, 0))
            ],
            core_axis_name=("c", "s"),
            dimension_semantics=(pltpu.PARALLEL,),
        )(idx_hbm, out_hbm)

    return gather_kernel(token_table, flat_ids.reshape(1, _N))


_RB = 800
_OB = _RB * 2 // _L


def _ln_body(tok_ref, pos_ref, mb_ref, gamma_ref, beta_ref, out_ref):
    y = tok_ref[...] + pos_ref[...]
    mb = mb_ref[...]
    dot = lambda a: lax.dot_general(
        a, mb, (((1,), (0,)), ((), ())),
        precision=lax.Precision.HIGHEST,
        preferred_element_type=jnp.float32,
    )
    m = dot(y)
    q = dot(y * y)
    r = lax.rsqrt(q - m * m + 1e-5)
    out_ref[...] = (y - m) * r * gamma_ref[...] + beta_ref[...]


def _tc_layernorm(tok2, pos_tiled, mb, g128, b128):
    return pl.pallas_call(
        _ln_body,
        grid=(_M // _RB,),
        in_specs=[
            pl.BlockSpec((_RB, 128), lambda i: (i, 0)),
            pl.BlockSpec((_RB, 128), lambda i: (0, 0)),
            pl.BlockSpec((128, 128), lambda i: (0, 0)),
            pl.BlockSpec((1, 128), lambda i: (0, 0)),
            pl.BlockSpec((1, 128), lambda i: (0, 0)),
        ],
        out_specs=pl.BlockSpec((_RB, 128), lambda i: (i, 0)),
        out_shape=jax.ShapeDtypeStruct((_M, 128), jnp.float32),
    )(tok2, pos_tiled, mb, g128, b128)


def kernel(input_ids, token_table, pos_table, gamma, beta):
    flat_ids = input_ids.reshape(-1).astype(jnp.int32)
    tok2 = _sc_gather(token_table, flat_ids).reshape(_M, 2 * _EMBED)

    pos_pairs = pos_table[:_L].reshape(_L // 2, 128)
    pos_tiled = jnp.tile(pos_pairs, (_RB * 2 // _L, 1))
    lane_grp = jnp.arange(128, dtype=jnp.int32) // _EMBED
    mb = jnp.where(lane_grp[:, None] == lane_grp[None, :],
                   jnp.float32(1.0 / _EMBED), jnp.float32(0.0))
    g128 = jnp.tile(gamma, 2).reshape(1, 128)
    b128 = jnp.tile(beta, 2).reshape(1, 128)
    out2 = _tc_layernorm(tok2, pos_tiled, mb, g128, b128)
    return out2.reshape(_B, _L, _EMBED)


# speedup vs baseline: 1.3086x; 1.3086x over previous
import functools

import jax
import jax.numpy as jnp
from jax import lax
from jax.experimental import pallas as pl
from jax.experimental.pallas import tpu as pltpu
from jax.experimental.pallas import tpu_sc as plsc

_VOCAB = 1000000
_EMBED = 64
_B = 4096
_L = 200
_N = _B * _L
_LP = _L // 2

_GATHER_WINDOW = 128


def _sc_gather(token_table, flat_ids):
    mesh = plsc.VectorSubcoreMesh(core_axis_name="c", subcore_axis_name="s")

    @functools.partial(
        pl.kernel,
        out_type=jax.ShapeDtypeStruct((_N, _EMBED), jnp.float32),
        mesh=mesh,
        compiler_params=pltpu.CompilerParams(use_tc_tiling_on_sc=False),
    )
    def gather_kernel(table_hbm, idx_hbm, out_hbm):
        def body(i_vmem, o_vmem):
            pltpu.sync_copy(table_hbm.at[i_vmem.at[0]], o_vmem)

        pltpu.emit_pipeline(
            body,
            grid=(_N // _GATHER_WINDOW,),
            in_specs=[
                pl.BlockSpec((1, _GATHER_WINDOW), index_map=lambda i: (0, i))
            ],
            out_specs=[
                pl.BlockSpec((_GATHER_WINDOW, _EMBED), index_map=lambda i: (i, 0))
            ],
            core_axis_name=("c", "s"),
            dimension_semantics=(pltpu.PARALLEL,),
        )(idx_hbm, out_hbm)

    return gather_kernel(token_table, flat_ids.reshape(1, _N))


_BP = 4
_BC = 512


def _ln_p_body(tok_ref, pos_ref, gamma_ref, beta_ref, out_ref):
    y = tok_ref[...] + pos_ref[...]
    g = gamma_ref[...]
    b = beta_ref[...]
    for h in (slice(0, _EMBED), slice(_EMBED, 128)):
        yh = y[:, h, :]
        m = jnp.mean(yh, axis=1, keepdims=True)
        q = jnp.mean(yh * yh, axis=1, keepdims=True)
        r = lax.rsqrt(q - m * m + 1e-5)
        out_ref[:, h, :] = (yh - m) * r * g[:, h, :] + b[:, h, :]


def _tc_layernorm_p(tokP, posP, gP, bP):
    return pl.pallas_call(
        _ln_p_body,
        grid=(_LP // _BP, _B // _BC),
        in_specs=[
            pl.BlockSpec((_BP, 128, _BC), lambda i, j: (i, 0, j)),
            pl.BlockSpec((_BP, 128, 1), lambda i, j: (i, 0, 0)),
            pl.BlockSpec((1, 128, 1), lambda i, j: (0, 0, 0)),
            pl.BlockSpec((1, 128, 1), lambda i, j: (0, 0, 0)),
        ],
        out_specs=pl.BlockSpec((_BP, 128, _BC), lambda i, j: (i, 0, j)),
        out_shape=jax.ShapeDtypeStruct((_LP, 128, _B), jnp.float32),
    )(tokP, posP, gP, bP)


def kernel(input_ids, token_table, pos_table, gamma, beta):
    flat_ids = input_ids.reshape(-1).astype(jnp.int32)
    tok2 = _sc_gather(token_table, flat_ids)
    tokP = jnp.transpose(tok2.reshape(_B, _LP, 128), (1, 2, 0))
    posP = pos_table[:_L].reshape(_LP, 128, 1)
    gP = jnp.tile(gamma, 2).reshape(1, 128, 1)
    bP = jnp.tile(beta, 2).reshape(1, 128, 1)
    outP = _tc_layernorm_p(tokP, posP, gP, bP)
    return jnp.transpose(outP.reshape(_L, _EMBED, _B), (2, 0, 1))
